# baseline (speedup 1.0000x reference)
import functools

import jax
import jax.numpy as jnp
from jax import lax
from jax.experimental import pallas as pl
from jax.experimental.pallas import tpu as pltpu
from jax.experimental.pallas import tpu_sc as plsc

B, L, V, S, D = 1024, 200, 100000, 2, 128

_info = plsc.get_sparse_core_info()
NC, NS, LN = _info.num_cores, _info.num_subcores, _info.num_lanes
NW = NC * NS
ROWS = B * L
RPW = ROWS // NW
G = 128
NG = RPW // G
KV = G // LN

_mesh = plsc.VectorSubcoreMesh(core_axis_name="c", subcore_axis_name="s")


@functools.partial(
    pl.kernel,
    mesh=_mesh,
    out_type=jax.ShapeDtypeStruct((ROWS, D), jnp.float32),
    scratch_types=[
        pltpu.VMEM((NG, G), jnp.int32),
        pltpu.VMEM((NG, G), jnp.int32),
        pltpu.VMEM((S * L, D), jnp.float32),
        pltpu.VMEM((G, D), jnp.float32),
        pltpu.VMEM((G, D), jnp.float32),
        pltpu.SemaphoreType.DMA,
        pltpu.SemaphoreType.DMA,
        pltpu.SemaphoreType.DMA,
        pltpu.SemaphoreType.DMA,
    ],
)
def _emb_kernel(tok_hbm, posseg_hbm, tokens_hbm, segments_hbm, out_hbm,
                tokidx, psidx, pslocal, tb0, tb1, st0, st1, so0, so1):
    wid = lax.axis_index("s") * NC + lax.axis_index("c")
    base = wid * RPW
    bufs = ((tb0, st0, so0), (tb1, st1, so1))

    pltpu.sync_copy(tokens_hbm.at[wid], tokidx)
    pltpu.sync_copy(segments_hbm.at[wid], psidx)
    pltpu.sync_copy(posseg_hbm, pslocal)

    iota = lax.iota(jnp.int32, LN)

    def prep_body(gg, _):
        for kk in range(KV):
            off = kk * LN
            s16 = psidx[gg, pl.ds(off, LN)]
            p = iota + (base + off) + gg * G
            l = lax.rem(p, L)
            psidx[gg, pl.ds(off, LN)] = 2 * l + s16
        return 0

    lax.fori_loop(0, NG, prep_body, 0)

    def issue_gather(g, b):
        tb, st, _ = bufs[b]
        pltpu.async_copy(tok_hbm.at[tokidx.at[g]], tb, st)

    issue_gather(0, 0)
    issue_gather(1, 1)

    def pair_body(i, _):
        for b in range(2):
            g = i * 2 + b
            tb, st, so = bufs[b]
            pltpu.make_async_copy(tok_hbm.at[tokidx.at[g]], tb, st).wait()

            def add_block(jj, _):
                ps16 = psidx[g, pl.ds(jj * LN, LN)]
                t16 = tokidx[g, pl.ds(jj * LN, LN)]
                for rr in range(LN):
                    r = jj * LN + rr
                    ps = ps16[rr]
                    m = jnp.where(t16[rr] == 0, 0.0, 1.0)
                    mv = lax.broadcast(m, (LN,))
                    for c in range(D // LN):
                        sl = pl.ds(c * LN, LN)
                        tb[r, sl] = tb[r, sl] * mv + pslocal[ps, sl]
                return 0

            lax.fori_loop(0, KV, add_block, 0)
            pltpu.async_copy(tb, out_hbm.at[pl.ds(base + g * G, G)], so)
            pltpu.make_async_copy(
                tb, out_hbm.at[pl.ds(base + g * G, G)], so).wait()

            @pl.when(g + 2 < NG)
            def _next():
                issue_gather(g + 2, b)
        return 0

    lax.fori_loop(0, NG // 2, pair_body, 0)


def kernel(tokens, segments, tok_table, pos_table, seg_table):
    seg_z = seg_table.at[0].set(0.0)
    posseg = (pos_table[:, None, :] + seg_z[None, :, :]).reshape(L * S, D)
    out = _emb_kernel(
        tok_table,
        posseg,
        tokens.reshape(NW, NG, G).astype(jnp.int32),
        segments.reshape(NW, NG, G).astype(jnp.int32),
    )
    return out.reshape(B, L, D)
